# baseline (speedup 1.0000x reference)
import jax
import jax.numpy as jnp
from jax.experimental import pallas as pl
from jax.experimental.pallas import tpu as pltpu

_SCORE_THRESH = 0.05
_NMS_THRESH = 0.5
_MAX_DET = 100
_N = 20000
_R = 160
_C = 128
_PAD = _R * _C
_NBLK = _R // 8
_NEG = -jnp.inf


def _nms_kernel(
    x1_ref, y1_ref, x2_ref, y2_ref, s_ref, out_ref,
    work_ref, area_ref, cmax_ref, irow_ref,
    ax1_ref, ay1_ref, ax2_ref, ay2_ref,
    sb_ref,
):
    x1 = x1_ref[...]
    y1 = y1_ref[...]
    x2 = x2_ref[...]
    y2 = y2_ref[...]
    area_ref[...] = (x2 - x1) * (y2 - y1)
    s = s_ref[...]
    work_ref[...] = jnp.where(s > _SCORE_THRESH, s, _NEG)

    lane = jax.lax.broadcasted_iota(jnp.int32, (1, _C), 1)
    blk_row = jax.lax.broadcasted_iota(jnp.int32, (8, _C), 0)

    def pass_a(cnt):
        def supj(j, carry):
            bx1 = sb_ref[j, 0]
            by1 = sb_ref[j, 1]
            bx2 = sb_ref[j, 2]
            by2 = sb_ref[j, 3]
            barea = (bx2 - bx1) * (by2 - by1)
            xx1 = jnp.maximum(x1, bx1)
            yy1 = jnp.maximum(y1, by1)
            xx2 = jnp.minimum(x2, bx2)
            yy2 = jnp.minimum(y2, by2)
            inter = jnp.maximum(xx2 - xx1, 0.0) * jnp.maximum(yy2 - yy1, 0.0)
            iou = inter / (area_ref[...] + barea - inter + 1e-9)
            work_ref[...] = jnp.where(iou > _NMS_THRESH, _NEG, work_ref[...])
            return carry

        jax.lax.fori_loop(0, cnt, supj, 0)
        neww = work_ref[...]

        m1 = jnp.full((8, _C), _NEG, dtype=jnp.float32)
        m2 = jnp.full((8, _C), _NEG, dtype=jnp.float32)
        i1 = jnp.zeros((8, _C), dtype=jnp.int32)
        for v in range(_NBLK):
            xv = neww[v * 8:(v + 1) * 8, :]
            gt = xv > m1
            m2 = jnp.maximum(m2, jnp.where(gt, m1, jnp.minimum(m1, xv)))
            i1 = jnp.where(gt, blk_row + v * 8, i1)
            m1 = jnp.where(gt, xv, m1)
        vmax = jnp.max(m1, axis=0, keepdims=True)
        eqm = m1 == vmax
        dup = jnp.sum(eqm.astype(jnp.int32), axis=0, keepdims=True) >= 2
        sec1 = jnp.max(jnp.where(eqm, _NEG, m1), axis=0, keepdims=True)
        c2 = jnp.maximum(
            jnp.max(m2, axis=0, keepdims=True),
            jnp.where(dup, vmax, sec1),
        )
        irow = jnp.min(jnp.where(eqm, i1, _PAD), axis=0, keepdims=True)
        cmax_ref[...] = vmax
        irow_ref[...] = irow
        ax1_ref[...] = jnp.zeros((1, _C), dtype=jnp.float32)
        ay1_ref[...] = jnp.zeros((1, _C), dtype=jnp.float32)
        ax2_ref[...] = jnp.zeros((1, _C), dtype=jnp.float32)
        ay2_ref[...] = jnp.zeros((1, _C), dtype=jnp.float32)
        return jnp.max(c2)

    def round_body(carry):
        cnt_prev, n_out = carry
        t = pass_a(cnt_prev)

        def drain_cond(c):
            _, n, go = c
            return go & (n < _MAX_DET)

        def drain_body(c):
            cnt, n, _ = c
            cm = cmax_ref[...]
            m = jnp.max(cm)
            first = cnt == 0
            ok_score = first | (m > t)
            valid = m != _NEG
            do_zero = ok_score & (~valid)
            do_cand = ok_score & valid
            flat = jnp.min(
                jnp.where(cm == m, irow_ref[...] * _C + lane, _PAD)
            )
            r = flat // _C
            cc = flat % _C
            sel = lane == cc
            rowx1 = jnp.max(jnp.where(sel, x1_ref[pl.ds(r, 1), :], _NEG))
            rowy1 = jnp.max(jnp.where(sel, y1_ref[pl.ds(r, 1), :], _NEG))
            rowx2 = jnp.max(jnp.where(sel, x2_ref[pl.ds(r, 1), :], _NEG))
            rowy2 = jnp.max(jnp.where(sel, y2_ref[pl.ds(r, 1), :], _NEG))
            a1 = ax1_ref[...]
            b1 = ay1_ref[...]
            a2 = ax2_ref[...]
            b2 = ay2_ref[...]
            xx1 = jnp.maximum(a1, rowx1)
            yy1 = jnp.maximum(b1, rowy1)
            xx2 = jnp.minimum(a2, rowx2)
            yy2 = jnp.minimum(b2, rowy2)
            inter = jnp.maximum(xx2 - xx1, 0.0) * jnp.maximum(yy2 - yy1, 0.0)
            aarea = (a2 - a1) * (b2 - b1)
            barea = (rowx2 - rowx1) * (rowy2 - rowy1)
            iou = inter / (aarea + barea - inter + 1e-9)
            sup = jnp.max(jnp.where(lane < cnt, iou, 0.0)) > _NMS_THRESH
            accept = do_cand & (~sup)

            @pl.when(do_cand)
            def _():
                cmax_ref[...] = jnp.where(sel, _NEG, cm)

            @pl.when(accept)
            def _():
                row = (
                    jnp.where(lane == 0, rowx1, 0.0)
                    + jnp.where(lane == 1, rowy1, 0.0)
                    + jnp.where(lane == 2, rowx2, 0.0)
                    + jnp.where(lane == 3, rowy2, 0.0)
                    + jnp.where(lane == 4, m, 0.0)
                )
                out_ref[pl.ds(n, 1), :] = row
                sb_ref[cnt, 0] = rowx1
                sb_ref[cnt, 1] = rowy1
                sb_ref[cnt, 2] = rowx2
                sb_ref[cnt, 3] = rowy2
                newlane = lane == cnt
                ax1_ref[...] = jnp.where(newlane, rowx1, a1)
                ay1_ref[...] = jnp.where(newlane, rowy1, b1)
                ax2_ref[...] = jnp.where(newlane, rowx2, a2)
                ay2_ref[...] = jnp.where(newlane, rowy2, b2)

            @pl.when(do_zero)
            def _():
                out_ref[pl.ds(n, 1), :] = jnp.zeros((1, _C), jnp.float32)

            new_cnt = jnp.where(accept, cnt + 1, cnt)
            new_n = jnp.where(accept | do_zero, n + 1, n)
            go = do_cand
            return (new_cnt, new_n, go)

        cnt, n_out, _ = jax.lax.while_loop(
            drain_cond, drain_body, (jnp.int32(0), n_out, True)
        )
        return (cnt, n_out)

    jax.lax.while_loop(
        lambda c: c[1] < _MAX_DET,
        round_body,
        (jnp.int32(0), jnp.int32(0)),
    )


def kernel(boxes, scores):
    pad_boxes = jnp.zeros((_PAD - _N, 4), dtype=boxes.dtype)
    b = jnp.concatenate([boxes, pad_boxes], axis=0)
    s = jnp.concatenate(
        [scores, jnp.full((_PAD - _N,), -1.0, dtype=scores.dtype)], axis=0
    ).reshape(_R, _C)
    x1 = b[:, 0].reshape(_R, _C)
    y1 = b[:, 1].reshape(_R, _C)
    x2 = b[:, 2].reshape(_R, _C)
    y2 = b[:, 3].reshape(_R, _C)
    out = pl.pallas_call(
        _nms_kernel,
        out_shape=jax.ShapeDtypeStruct((_MAX_DET, _C), jnp.float32),
        scratch_shapes=[
            pltpu.VMEM((_R, _C), jnp.float32),
            pltpu.VMEM((_R, _C), jnp.float32),
            pltpu.VMEM((1, _C), jnp.float32),
            pltpu.VMEM((1, _C), jnp.int32),
            pltpu.VMEM((1, _C), jnp.float32),
            pltpu.VMEM((1, _C), jnp.float32),
            pltpu.VMEM((1, _C), jnp.float32),
            pltpu.VMEM((1, _C), jnp.float32),
            pltpu.SMEM((_MAX_DET, 4), jnp.float32),
        ],
    )(x1, y1, x2, y2, s)
    return out[:, :5]
